# Initial kernel scaffold; baseline (speedup 1.0000x reference)
import jax
import jax.numpy as jnp
from jax.experimental import pallas as pl


def kernel(x, edge_index, W_l1, b_l1, W_r1, W_l2, b_l2, W_r2):
    raise NotImplementedError("write your pallas kernel here")


# speedup vs baseline: 4.0938x; 4.0938x over previous
import functools

import jax
import jax.numpy as jnp
from jax import lax
from jax.experimental import pallas as pl
from jax.experimental.pallas import tpu as pltpu
from jax.experimental.pallas import tpu_sc as plsc

N = 10000
D = 128
E = 320000

NC = 2
NS = 16
FS = D // NC

CH = 128
NCH = 157
EPW = CH * NCH
E_PAD = EPW * NS
N_PAD = 10240
RPS = N_PAD // NS

_mesh = plsc.VectorSubcoreMesh(core_axis_name="c", subcore_axis_name="s")


@functools.partial(
    pl.kernel,
    mesh=_mesh,
    out_type=[
        jax.ShapeDtypeStruct((NC, N_PAD, FS), jnp.float32),
        jax.ShapeDtypeStruct((NC, N_PAD, 16), jnp.float32),
    ],
    scratch_types=[
        pltpu.VMEM((CH,), jnp.int32),
        pltpu.VMEM((CH,), jnp.int32),
        pltpu.VMEM((CH, FS), jnp.float32),
        pltpu.VMEM((CH, 16), jnp.float32),
        pltpu.VMEM((RPS, 16), jnp.float32),
        pltpu.VMEM_SHARED((N_PAD, FS), jnp.float32),
        pltpu.VMEM_SHARED((N_PAD, 16), jnp.float32),
        pltpu.SemaphoreType.DMA,
    ],
    compiler_params=pltpu.CompilerParams(use_tc_tiling_on_sc=False),
)
def _sc_aggregate(feat_hbm, src_hbm, dst_hbm, zfeat_hbm, zdeg_hbm, ones_hbm,
                  acc_hbm, deg_hbm,
                  src_v, dst_v, rows_v, ones_v, dstage_v, acc_sh, deg_sh,
                  sem):
    c = lax.axis_index("c")
    s = lax.axis_index("s")

    pltpu.sync_copy(zfeat_hbm, rows_v)
    for k in range(RPS // CH):
        pltpu.sync_copy(rows_v, acc_sh.at[pl.ds(s * RPS + k * CH, CH)])
    pltpu.sync_copy(zdeg_hbm, dstage_v)
    pltpu.sync_copy(dstage_v, deg_sh.at[pl.ds(s * RPS, RPS)])
    pltpu.sync_copy(ones_hbm, ones_v)
    plsc.subcore_barrier()

    fh = feat_hbm.at[c]

    def chunk_body(j, carry):
        base = s * EPW + j * CH
        pltpu.sync_copy(src_hbm.at[pl.ds(base, CH)], src_v)
        pltpu.sync_copy(dst_hbm.at[pl.ds(base, CH)], dst_v)
        pltpu.async_copy(fh.at[src_v], rows_v, sem).wait()
        pltpu.sync_copy(rows_v, acc_sh.at[dst_v], add=True)
        pltpu.sync_copy(ones_v, deg_sh.at[dst_v], add=True)
        return carry

    lax.fori_loop(0, NCH, chunk_body, 0)

    plsc.subcore_barrier()
    for k in range(RPS // CH):
        pltpu.sync_copy(acc_sh.at[pl.ds(s * RPS + k * CH, CH)], rows_v)
        pltpu.sync_copy(rows_v, acc_hbm.at[c, pl.ds(s * RPS + k * CH, CH)])
    pltpu.sync_copy(deg_sh.at[pl.ds(s * RPS, RPS)], dstage_v)
    pltpu.sync_copy(dstage_v, deg_hbm.at[c, pl.ds(s * RPS, RPS)])


def _combine_body(relu, acc_ref, degc_ref, xs_ref, wl_ref, wr_ref, b_ref,
                  o_ref, os_ref):
    deg = degc_ref[0, :, 0]
    scale = 1.0 / jnp.maximum(deg, 1.0)
    mean = jnp.concatenate(
        [acc_ref[0, :, :] * scale[:, None],
         acc_ref[1, :, :] * scale[:, None]], axis=1)
    xfull = jnp.concatenate([xs_ref[0, :, :], xs_ref[1, :, :]], axis=1)
    out = (jnp.dot(mean, wl_ref[...], preferred_element_type=jnp.float32)
           + jnp.dot(xfull, wr_ref[...],
                     preferred_element_type=jnp.float32)
           + b_ref[...])
    if relu:
        out = jnp.maximum(out, 0.0)
    o_ref[...] = out
    os_ref[0, :, :] = out[:, :FS]
    os_ref[1, :, :] = out[:, FS:]


def _tc_combine(acc, degc, xs, W_l, W_r, b, relu):
    R = 512
    grid = N_PAD // R
    return pl.pallas_call(
        functools.partial(_combine_body, relu),
        grid=(grid,),
        in_specs=[
            pl.BlockSpec((NC, R, FS), lambda i: (0, i, 0)),
            pl.BlockSpec((NC, R, 16), lambda i: (0, i, 0)),
            pl.BlockSpec((NC, R, FS), lambda i: (0, i, 0)),
            pl.BlockSpec((D, D), lambda i: (0, 0)),
            pl.BlockSpec((D, D), lambda i: (0, 0)),
            pl.BlockSpec((1, D), lambda i: (0, 0)),
        ],
        out_specs=[
            pl.BlockSpec((R, D), lambda i: (i, 0)),
            pl.BlockSpec((NC, R, FS), lambda i: (0, i, 0)),
        ],
        out_shape=[
            jax.ShapeDtypeStruct((N_PAD, D), jnp.float32),
            jax.ShapeDtypeStruct((NC, N_PAD, FS), jnp.float32),
        ],
    )(acc, degc, xs, W_l, W_r, b.reshape(1, D))


def kernel(x, edge_index, W_l1, b_l1, W_r1, W_l2, b_l2, W_r2):
    src = edge_index[0]
    dst = edge_index[1]
    src_p = jnp.concatenate([src, jnp.zeros((E_PAD - E,), jnp.int32)])
    dst_p = jnp.concatenate([dst, jnp.full((E_PAD - E,), N, jnp.int32)])
    x_p = jnp.pad(x, ((0, N_PAD - N), (0, 0)))
    xs = jnp.stack([x_p[:, :FS], x_p[:, FS:]])
    zfeat = jnp.zeros((CH, FS), jnp.float32)
    zdeg = jnp.zeros((RPS, 16), jnp.float32)
    ones = jnp.ones((CH, 16), jnp.float32)

    acc1, degc = _sc_aggregate(xs, src_p, dst_p, zfeat, zdeg, ones)
    _, hs = _tc_combine(acc1, degc, xs, W_l1, W_r1, b_l1, relu=True)
    acc2, _ = _sc_aggregate(hs, src_p, dst_p, zfeat, zdeg, ones)
    out, _ = _tc_combine(acc2, degc, hs, W_l2, W_r2, b_l2, relu=False)
    return out[:N]


# speedup vs baseline: 4.4171x; 1.0790x over previous
import functools

import jax
import jax.numpy as jnp
from jax import lax
from jax.experimental import pallas as pl
from jax.experimental.pallas import tpu as pltpu
from jax.experimental.pallas import tpu_sc as plsc

N = 10000
D = 128
E = 320000

NC = 2
NS = 16
FS = D // NC

CH = 128
K = 4
NCH = 160
NG = NCH // K
EPW = CH * NCH
E_PAD = EPW * NS
TOTCH = E_PAD // CH
N_PAD = 10240
RPS = N_PAD // NS

_mesh = plsc.VectorSubcoreMesh(core_axis_name="c", subcore_axis_name="s")


def _make_sc_aggregate(with_deg):
    out_type = [jax.ShapeDtypeStruct((NC, N_PAD, FS), jnp.float32)]
    scratch_types = [
        pltpu.VMEM((K, 2, CH), jnp.int32),
        [pltpu.VMEM((CH, FS), jnp.float32) for _ in range(K)],
        pltpu.VMEM_SHARED((N_PAD, FS), jnp.float32),
        [pltpu.SemaphoreType.DMA for _ in range(K)],
        [pltpu.SemaphoreType.DMA for _ in range(K)],
    ]
    if with_deg:
        out_type.append(jax.ShapeDtypeStruct((NC, N_PAD, 16), jnp.float32))
        scratch_types.extend([
            pltpu.VMEM((CH, 16), jnp.float32),
            pltpu.VMEM((RPS, 16), jnp.float32),
            pltpu.VMEM_SHARED((N_PAD, 16), jnp.float32),
            [pltpu.SemaphoreType.DMA for _ in range(K)],
        ])

    @functools.partial(pl.kernel, mesh=_mesh, out_type=out_type,
                       scratch_types=scratch_types,
                       compiler_params=pltpu.CompilerParams(
                           use_tc_tiling_on_sc=False))
    def _sc_aggregate(feat_hbm, pe_hbm, zfeat_hbm, zdeg_hbm, ones_hbm,
                      *refs):
        if with_deg:
            (acc_hbm, deg_hbm, pidx_v, rows_v, acc_sh, gsem, ssem,
             ones_v, dstage_v, deg_sh, osem) = refs
        else:
            acc_hbm, pidx_v, rows_v, acc_sh, gsem, ssem = refs
        c = lax.axis_index("c")
        s = lax.axis_index("s")

        pltpu.sync_copy(zfeat_hbm, rows_v[0])
        for k in range(RPS // CH):
            pltpu.sync_copy(rows_v[0],
                            acc_sh.at[pl.ds(s * RPS + k * CH, CH)])
        if with_deg:
            pltpu.sync_copy(zdeg_hbm, dstage_v)
            pltpu.sync_copy(dstage_v, deg_sh.at[pl.ds(s * RPS, RPS)])
            pltpu.sync_copy(ones_hbm, ones_v)
        plsc.subcore_barrier()

        fh = feat_hbm.at[c]

        def group_body(g, carry):
            row = s * NCH + g * K
            pltpu.sync_copy(pe_hbm.at[pl.ds(row, K)], pidx_v)
            gathers = [
                pltpu.async_copy(fh.at[pidx_v.at[k, 0]], rows_v[k],
                                 gsem[k])
                for k in range(K)
            ]
            scatters = []
            for k in range(K):
                gathers[k].wait()
                scatters.append(
                    pltpu.async_copy(rows_v[k],
                                     acc_sh.at[pidx_v.at[k, 1]],
                                     ssem[k], add=True))
                if with_deg:
                    scatters.append(
                        pltpu.async_copy(ones_v,
                                         deg_sh.at[pidx_v.at[k, 1]],
                                         osem[k], add=True))
            for sc in scatters:
                sc.wait()
            return carry

        lax.fori_loop(0, NG, group_body, 0)

        plsc.subcore_barrier()
        for k in range(RPS // CH):
            pltpu.sync_copy(acc_sh.at[pl.ds(s * RPS + k * CH, CH)],
                            rows_v[0])
            pltpu.sync_copy(rows_v[0],
                            acc_hbm.at[c, pl.ds(s * RPS + k * CH, CH)])
        if with_deg:
            pltpu.sync_copy(deg_sh.at[pl.ds(s * RPS, RPS)], dstage_v)
            pltpu.sync_copy(dstage_v, deg_hbm.at[c, pl.ds(s * RPS, RPS)])

    return _sc_aggregate


_sc_aggregate_deg = _make_sc_aggregate(True)
_sc_aggregate_nodeg = _make_sc_aggregate(False)


def _combine_body(relu, acc_ref, degc_ref, xs_ref, wl_ref, wr_ref, b_ref,
                  o_ref, os_ref):
    deg = degc_ref[0, :, 0]
    scale = 1.0 / jnp.maximum(deg, 1.0)
    mean = jnp.concatenate(
        [acc_ref[0, :, :] * scale[:, None],
         acc_ref[1, :, :] * scale[:, None]], axis=1)
    xfull = jnp.concatenate([xs_ref[0, :, :], xs_ref[1, :, :]], axis=1)
    out = (jnp.dot(mean, wl_ref[...], preferred_element_type=jnp.float32)
           + jnp.dot(xfull, wr_ref[...],
                     preferred_element_type=jnp.float32)
           + b_ref[...])
    if relu:
        out = jnp.maximum(out, 0.0)
    o_ref[...] = out
    os_ref[0, :, :] = out[:, :FS]
    os_ref[1, :, :] = out[:, FS:]


def _tc_combine(acc, degc, xs, W_l, W_r, b, relu):
    R = 512
    grid = N_PAD // R
    return pl.pallas_call(
        functools.partial(_combine_body, relu),
        grid=(grid,),
        in_specs=[
            pl.BlockSpec((NC, R, FS), lambda i: (0, i, 0)),
            pl.BlockSpec((NC, R, 16), lambda i: (0, i, 0)),
            pl.BlockSpec((NC, R, FS), lambda i: (0, i, 0)),
            pl.BlockSpec((D, D), lambda i: (0, 0)),
            pl.BlockSpec((D, D), lambda i: (0, 0)),
            pl.BlockSpec((1, D), lambda i: (0, 0)),
        ],
        out_specs=[
            pl.BlockSpec((R, D), lambda i: (i, 0)),
            pl.BlockSpec((NC, R, FS), lambda i: (0, i, 0)),
        ],
        out_shape=[
            jax.ShapeDtypeStruct((N_PAD, D), jnp.float32),
            jax.ShapeDtypeStruct((NC, N_PAD, FS), jnp.float32),
        ],
    )(acc, degc, xs, W_l, W_r, b.reshape(1, D))


def kernel(x, edge_index, W_l1, b_l1, W_r1, W_l2, b_l2, W_r2):
    src = edge_index[0]
    dst = edge_index[1]
    src_p = jnp.concatenate([src, jnp.zeros((E_PAD - E,), jnp.int32)])
    dst_p = jnp.concatenate([dst, jnp.full((E_PAD - E,), N, jnp.int32)])
    pe = jnp.stack([src_p.reshape(TOTCH, CH), dst_p.reshape(TOTCH, CH)],
                   axis=1)
    x_p = jnp.pad(x, ((0, N_PAD - N), (0, 0)))
    xs = jnp.stack([x_p[:, :FS], x_p[:, FS:]])
    zfeat = jnp.zeros((CH, FS), jnp.float32)
    zdeg = jnp.zeros((RPS, 16), jnp.float32)
    ones = jnp.ones((CH, 16), jnp.float32)

    acc1, degc = _sc_aggregate_deg(xs, pe, zfeat, zdeg, ones)
    _, hs = _tc_combine(acc1, degc, xs, W_l1, W_r1, b_l1, relu=True)
    (acc2,) = _sc_aggregate_nodeg(hs, pe, zfeat, zdeg, ones)
    out, _ = _tc_combine(acc2, degc, hs, W_l2, W_r2, b_l2, relu=False)
    return out[:N]


# speedup vs baseline: 4.7083x; 1.0659x over previous
import functools

import jax
import jax.numpy as jnp
from jax import lax
from jax.experimental import pallas as pl
from jax.experimental.pallas import tpu as pltpu
from jax.experimental.pallas import tpu_sc as plsc

N = 10000
D = 128
E = 320000

NC = 2
NS = 16
FS = D // NC

CH = 128
K = 8
NCH = 160
NG = NCH // K
EPW = CH * NCH
E_PAD = EPW * NS
TOTCH = E_PAD // CH
N_PAD = 10240
RPS = N_PAD // NS

_mesh = plsc.VectorSubcoreMesh(core_axis_name="c", subcore_axis_name="s")


def _make_sc_aggregate(with_deg):
    out_type = [jax.ShapeDtypeStruct((NC, N_PAD, FS), jnp.float32)]
    scratch_types = [
        pltpu.VMEM((K, 2, CH), jnp.int32),
        [pltpu.VMEM((CH, FS), jnp.float32) for _ in range(K)],
        pltpu.VMEM_SHARED((N_PAD, FS), jnp.float32),
        [pltpu.SemaphoreType.DMA for _ in range(K)],
        [pltpu.SemaphoreType.DMA for _ in range(K)],
    ]
    if with_deg:
        out_type.append(jax.ShapeDtypeStruct((NC, N_PAD, 16), jnp.float32))
        scratch_types.extend([
            pltpu.VMEM((CH, 16), jnp.float32),
            pltpu.VMEM((RPS, 16), jnp.float32),
            pltpu.VMEM_SHARED((N_PAD, 16), jnp.float32),
            [pltpu.SemaphoreType.DMA for _ in range(K)],
        ])

    @functools.partial(pl.kernel, mesh=_mesh, out_type=out_type,
                       scratch_types=scratch_types,
                       compiler_params=pltpu.CompilerParams(
                           use_tc_tiling_on_sc=False))
    def _sc_aggregate(feat_hbm, pe_hbm, zfeat_hbm, zdeg_hbm, ones_hbm,
                      *refs):
        if with_deg:
            (acc_hbm, deg_hbm, pidx_v, rows_v, acc_sh, gsem, ssem,
             ones_v, dstage_v, deg_sh, osem) = refs
        else:
            acc_hbm, pidx_v, rows_v, acc_sh, gsem, ssem = refs
        c = lax.axis_index("c")
        s = lax.axis_index("s")

        pltpu.sync_copy(zfeat_hbm, rows_v[0])
        for k in range(RPS // CH):
            pltpu.sync_copy(rows_v[0],
                            acc_sh.at[pl.ds(s * RPS + k * CH, CH)])
        if with_deg:
            pltpu.sync_copy(zdeg_hbm, dstage_v)
            pltpu.sync_copy(dstage_v, deg_sh.at[pl.ds(s * RPS, RPS)])
            pltpu.sync_copy(ones_hbm, ones_v)
        plsc.subcore_barrier()

        fh = feat_hbm.at[c]

        def group_body(g, carry):
            row = s * NCH + g * K
            pltpu.sync_copy(pe_hbm.at[pl.ds(row, K)], pidx_v)
            gathers = [
                pltpu.async_copy(fh.at[pidx_v.at[k, 0]], rows_v[k],
                                 gsem[k])
                for k in range(K)
            ]
            scatters = []
            for k in range(K):
                gathers[k].wait()
                scatters.append(
                    pltpu.async_copy(rows_v[k],
                                     acc_sh.at[pidx_v.at[k, 1]],
                                     ssem[k], add=True))
                if with_deg:
                    scatters.append(
                        pltpu.async_copy(ones_v,
                                         deg_sh.at[pidx_v.at[k, 1]],
                                         osem[k], add=True))
            for sc in scatters:
                sc.wait()
            return carry

        lax.fori_loop(0, NG, group_body, 0)

        plsc.subcore_barrier()
        for k in range(RPS // CH):
            pltpu.sync_copy(acc_sh.at[pl.ds(s * RPS + k * CH, CH)],
                            rows_v[0])
            pltpu.sync_copy(rows_v[0],
                            acc_hbm.at[c, pl.ds(s * RPS + k * CH, CH)])
        if with_deg:
            pltpu.sync_copy(deg_sh.at[pl.ds(s * RPS, RPS)], dstage_v)
            pltpu.sync_copy(dstage_v, deg_hbm.at[c, pl.ds(s * RPS, RPS)])

    return _sc_aggregate


_sc_aggregate_deg = _make_sc_aggregate(True)
_sc_aggregate_nodeg = _make_sc_aggregate(False)


def _combine_body(relu, acc_ref, degc_ref, xs_ref, wl_ref, wr_ref, b_ref,
                  o_ref, os_ref):
    deg = degc_ref[0, :, 0]
    scale = 1.0 / jnp.maximum(deg, 1.0)
    mean = jnp.concatenate(
        [acc_ref[0, :, :] * scale[:, None],
         acc_ref[1, :, :] * scale[:, None]], axis=1)
    xfull = jnp.concatenate([xs_ref[0, :, :], xs_ref[1, :, :]], axis=1)
    out = (jnp.dot(mean, wl_ref[...], preferred_element_type=jnp.float32)
           + jnp.dot(xfull, wr_ref[...],
                     preferred_element_type=jnp.float32)
           + b_ref[...])
    if relu:
        out = jnp.maximum(out, 0.0)
    o_ref[...] = out
    os_ref[0, :, :] = out[:, :FS]
    os_ref[1, :, :] = out[:, FS:]


def _tc_combine(acc, degc, xs, W_l, W_r, b, relu):
    R = 512
    grid = N_PAD // R
    return pl.pallas_call(
        functools.partial(_combine_body, relu),
        grid=(grid,),
        in_specs=[
            pl.BlockSpec((NC, R, FS), lambda i: (0, i, 0)),
            pl.BlockSpec((NC, R, 16), lambda i: (0, i, 0)),
            pl.BlockSpec((NC, R, FS), lambda i: (0, i, 0)),
            pl.BlockSpec((D, D), lambda i: (0, 0)),
            pl.BlockSpec((D, D), lambda i: (0, 0)),
            pl.BlockSpec((1, D), lambda i: (0, 0)),
        ],
        out_specs=[
            pl.BlockSpec((R, D), lambda i: (i, 0)),
            pl.BlockSpec((NC, R, FS), lambda i: (0, i, 0)),
        ],
        out_shape=[
            jax.ShapeDtypeStruct((N_PAD, D), jnp.float32),
            jax.ShapeDtypeStruct((NC, N_PAD, FS), jnp.float32),
        ],
    )(acc, degc, xs, W_l, W_r, b.reshape(1, D))


def kernel(x, edge_index, W_l1, b_l1, W_r1, W_l2, b_l2, W_r2):
    src = edge_index[0]
    dst = edge_index[1]
    src_p = jnp.concatenate([src, jnp.zeros((E_PAD - E,), jnp.int32)])
    dst_p = jnp.concatenate([dst, jnp.full((E_PAD - E,), N, jnp.int32)])
    pe = jnp.stack([src_p.reshape(TOTCH, CH), dst_p.reshape(TOTCH, CH)],
                   axis=1)
    x_p = jnp.pad(x, ((0, N_PAD - N), (0, 0)))
    xs = jnp.stack([x_p[:, :FS], x_p[:, FS:]])
    zfeat = jnp.zeros((CH, FS), jnp.float32)
    zdeg = jnp.zeros((RPS, 16), jnp.float32)
    ones = jnp.ones((CH, 16), jnp.float32)

    acc1, degc = _sc_aggregate_deg(xs, pe, zfeat, zdeg, ones)
    _, hs = _tc_combine(acc1, degc, xs, W_l1, W_r1, b_l1, relu=True)
    (acc2,) = _sc_aggregate_nodeg(hs, pe, zfeat, zdeg, ones)
    out, _ = _tc_combine(acc2, degc, hs, W_l2, W_r2, b_l2, relu=False)
    return out[:N]
